# Initial kernel scaffold; baseline (speedup 1.0000x reference)
import jax
import jax.numpy as jnp
from jax.experimental import pallas as pl


def kernel(x, emb_table):
    raise NotImplementedError("write your pallas kernel here")


# speedup vs baseline: 3.1410x; 3.1410x over previous
import functools

import jax
import jax.numpy as jnp
from jax import lax
from jax.experimental import pallas as pl
from jax.experimental.pallas import tpu as pltpu


def _round_up(n, m):
    return ((n + m - 1) // m) * m


def _count_embed_kernel(x_ref, emb_ref, out_ref, cnt_ref, *, inv_len,
                        num_full_tiles, vocab_tile, tail_rows):
    x = x_ref[...]
    L, _, TB = x.shape
    TV = vocab_tile

    for k in range(num_full_tiles):
        ids = lax.broadcasted_iota(jnp.int32, (L, TV, TB), 1) + (k * TV)
        onehot = (x == ids).astype(jnp.float32)
        cnt_ref[k * TV:(k + 1) * TV, :] = jnp.sum(onehot, axis=0)

    base = num_full_tiles * TV
    ids = lax.broadcasted_iota(jnp.int32, (L, tail_rows, TB), 1) + base
    onehot = (x == ids).astype(jnp.float32)
    cnt_ref[base:base + tail_rows, :] = jnp.sum(onehot, axis=0)

    acc = lax.dot_general(cnt_ref[...], emb_ref[...],
                          (((0,), (0,)), ((), ())),
                          preferred_element_type=jnp.float32)
    out_ref[...] = (acc * jnp.float32(inv_len)).astype(out_ref.dtype)


@functools.partial(jax.jit, static_argnames=("batch_tile",))
def _count_morgan_embedding(x, emb_table, batch_tile=128):
    B, L = x.shape
    V, D = emb_table.shape

    TB = batch_tile
    TV = 128
    num_full_tiles = V // TV
    tail = V - num_full_tiles * TV
    tail_rows = _round_up(max(tail, 1), 8)
    V_pad = num_full_tiles * TV + tail_rows
    D_pad = _round_up(D, 128)

    x_t = jnp.transpose(x.astype(jnp.int32)).reshape(L, 1, B)
    emb_p = jnp.pad(emb_table.astype(jnp.float32),
                    ((0, V_pad - V), (0, D_pad - D)))

    out = pl.pallas_call(
        functools.partial(_count_embed_kernel, inv_len=1.0 / L,
                          num_full_tiles=num_full_tiles, vocab_tile=TV,
                          tail_rows=tail_rows),
        out_shape=jax.ShapeDtypeStruct((B, D_pad), jnp.float32),
        grid_spec=pltpu.PrefetchScalarGridSpec(
            num_scalar_prefetch=0,
            grid=(B // TB,),
            in_specs=[
                pl.BlockSpec((L, 1, TB), lambda i: (0, 0, i)),
                pl.BlockSpec((V_pad, D_pad), lambda i: (0, 0)),
            ],
            out_specs=pl.BlockSpec((TB, D_pad), lambda i: (i, 0)),
            scratch_shapes=[pltpu.VMEM((V_pad, TB), jnp.float32)],
        ),
        compiler_params=pltpu.CompilerParams(
            dimension_semantics=("parallel",)),
    )(x_t, emb_p)

    return out[:, :D].astype(emb_table.dtype)


def kernel(x, emb_table):
    return _count_morgan_embedding(x, emb_table)
